# Initial kernel scaffold; baseline (speedup 1.0000x reference)
import jax
import jax.numpy as jnp
from jax.experimental import pallas as pl


def kernel(x, z, edge_index, batch, target, params):
    raise NotImplementedError("write your pallas kernel here")


# speedup vs baseline: 3.1076x; 3.1076x over previous
import jax
import jax.numpy as jnp
from jax.experimental import pallas as pl
from jax.experimental.pallas import tpu as pltpu


def _mm_thresh(a, b, tm=512, tn=512):
    n, kdim = a.shape
    m = b.shape[1]

    def kern(a_ref, b_ref, o_ref, cs_ref):
        acc = jnp.dot(a_ref[...], b_ref[...], preferred_element_type=jnp.float32)
        t = (acc > 0.0).astype(jnp.float32)
        o_ref[...] = t.astype(jnp.bfloat16)
        part = jnp.sum(t, axis=0, keepdims=True)

        @pl.when(pl.program_id(1) == 0)
        def _():
            cs_ref[...] = part

        @pl.when(pl.program_id(1) != 0)
        def _():
            cs_ref[...] += part

    return pl.pallas_call(
        kern,
        grid=(m // tn, n // tm),
        in_specs=[
            pl.BlockSpec((tm, kdim), lambda j, i: (i, 0)),
            pl.BlockSpec((kdim, tn), lambda j, i: (0, j)),
        ],
        out_specs=[
            pl.BlockSpec((tm, tn), lambda j, i: (i, j)),
            pl.BlockSpec((1, tn), lambda j, i: (0, j)),
        ],
        out_shape=[
            jax.ShapeDtypeStruct((n, m), jnp.bfloat16),
            jax.ShapeDtypeStruct((1, m), jnp.float32),
        ],
    )(a, b)


def _gcn_apply(A, xw, dinv, bias, tm=512, tk=512):
    n = A.shape[0]
    f = xw.shape[1]
    nk = n // tk
    d2 = dinv.reshape(1, n)
    b2 = bias.reshape(1, f)

    def kern(a_ref, s_ref, di_ref, dk_ref, xi_ref, b_ref, o_ref, acc_ref):
        k = pl.program_id(1)

        @pl.when(k == 0)
        def _():
            acc_ref[...] = jnp.zeros_like(acc_ref)

        a = a_ref[...].astype(jnp.float32)
        s = s_ref[...] * dk_ref[0, :][:, None]
        acc_ref[...] += jax.lax.dot_general(
            a, s, (((0,), (0,)), ((), ())), preferred_element_type=jnp.float32)

        @pl.when(k == nk - 1)
        def _():
            di = di_ref[0, :][:, None]
            o_ref[...] = jnp.maximum(
                di * acc_ref[...] + (di * di) * xi_ref[...] + b_ref[...], 0.0)

    return pl.pallas_call(
        kern,
        grid=(n // tm, nk),
        in_specs=[
            pl.BlockSpec((tk, tm), lambda i, k: (k, i)),
            pl.BlockSpec((tk, f), lambda i, k: (k, 0)),
            pl.BlockSpec((1, tm), lambda i, k: (0, i)),
            pl.BlockSpec((1, tk), lambda i, k: (0, k)),
            pl.BlockSpec((tm, f), lambda i, k: (i, 0)),
            pl.BlockSpec((1, f), lambda i, k: (0, 0)),
        ],
        out_specs=pl.BlockSpec((tm, f), lambda i, k: (i, 0)),
        out_shape=jax.ShapeDtypeStruct((n, f), jnp.float32),
        scratch_shapes=[pltpu.VMEM((tm, f), jnp.float32)],
    )(A, xw, d2, d2, xw, b2)


def _padf(m):
    pad = (-m.shape[-1]) % 128
    if pad == 0:
        return m
    if m.ndim == 1:
        return jnp.pad(m, (0, pad))
    return jnp.pad(m, ((0, 0), (0, pad)))


def _bn(h):
    m = h.mean(axis=0)
    v = h.var(axis=0)
    return (h - m) / jnp.sqrt(v + 1e-5)


def kernel(x, z, edge_index, batch, target, params):
    p = params
    n = x.shape[0]
    nb = target.shape[0]
    relu = jax.nn.relu

    zc = jnp.concatenate([z, x], axis=-1)
    h = relu(_bn(zc @ p['vae_W0'] + p['vae_b0']))
    h = _bn(h @ p['vae_W1'] + p['vae_b1'])
    aug = jnp.tanh(h)
    X = jnp.concatenate([aug, x], axis=1)

    src = edge_index[0]
    dst = edge_index[1]
    Wcnt = jnp.zeros((n, n), jnp.float32).at[src, dst].add(1.0)
    adj = (Wcnt > 0).astype(jnp.bfloat16)
    Wcnt = Wcnt.astype(jnp.bfloat16)
    deg_e = jnp.sum(Wcnt, axis=0, dtype=jnp.float32) + 1.0
    dinv_e = jax.lax.rsqrt(jnp.maximum(deg_e, 1.0))

    h1 = _gcn_apply(Wcnt, _padf(X @ p['gcn_W1']), dinv_e, _padf(p['gcn_b1']))[:, :156]
    h2 = _gcn_apply(Wcnt, _padf(h1 @ p['gcn_W2']), dinv_e, _padf(p['gcn_b2']))[:, :312]
    h3 = _gcn_apply(Wcnt, _padf(h2 @ p['gcn_W3']), dinv_e, _padf(p['gcn_b3']))[:, :624]

    a2, cs2 = _mm_thresh(adj, adj)
    dd2 = jax.lax.rsqrt(cs2[0] + 1.0)
    a3, cs3 = _mm_thresh(a2, adj)
    dd3 = jax.lax.rsqrt(cs3[0] + 1.0)

    xw1 = _padf(X @ p['gcn_W1'])
    h4 = _gcn_apply(a2, xw1, dd2, _padf(p['gcn_b1']))[:, :156]
    h5 = _gcn_apply(a2, _padf(h4 @ p['gcn_W2']), dd2, _padf(p['gcn_b2']))[:, :312]
    h6 = _gcn_apply(a3, xw1, dd3, _padf(p['gcn_b1']))[:, :156]

    concat = jnp.concatenate([h3, h5, h6], axis=1)
    xg = jax.ops.segment_max(concat, batch, num_segments=nb)
    xg = relu(xg @ p['fcg1_W'] + p['fcg1_b'])
    xg = xg @ p['fcg2_W'] + p['fcg2_b']
    beta = jax.nn.softmax(jnp.tanh(xg), axis=1)
    xg = beta * xg

    emb = jnp.take(p['emb'], target, axis=0)
    e = jnp.transpose(emb, (0, 2, 1))

    def conv1d(xin, W, b):
        y = jax.lax.conv_general_dilated(
            xin, W, window_strides=(1,), padding='VALID',
            dimension_numbers=('NCH', 'OIH', 'NCH'))
        return relu(y + b[None, :, None])

    feats = []
    idx = 0
    for blk in range(3):
        hh = e
        for _ in range(blk + 1):
            hh = conv1d(hh, p['conv_W%d' % idx], p['conv_b%d' % idx])
            idx += 1
        feats.append(hh.max(axis=-1))
    xt = jnp.concatenate(feats, axis=-1) @ p['prot_W'] + p['prot_b']
    beta2 = jax.nn.softmax(jnp.tanh(xt), axis=1)
    xt = beta2 * xt

    xc = jnp.concatenate([xg, xt], axis=1)
    xc = relu(xc @ p['fc1_W'] + p['fc1_b'])
    xc = relu(xc @ p['fc2_W'] + p['fc2_b'])
    return xc @ p['out_W'] + p['out_b']


# speedup vs baseline: 3.4021x; 1.0948x over previous
import jax
import jax.numpy as jnp
from jax.experimental import pallas as pl
from jax.experimental.pallas import tpu as pltpu


def _mm_thresh(a, b, tm=1024, tn=1024):
    n, kdim = a.shape
    m = b.shape[1]

    def kern(a_ref, b_ref, o_ref, cs_ref):
        acc = jnp.dot(a_ref[...], b_ref[...], preferred_element_type=jnp.float32)
        t = (acc > 0.0).astype(jnp.float32)
        o_ref[...] = t.astype(jnp.bfloat16)
        part = jnp.sum(t, axis=0, keepdims=True)

        @pl.when(pl.program_id(1) == 0)
        def _():
            cs_ref[...] = part

        @pl.when(pl.program_id(1) != 0)
        def _():
            cs_ref[...] += part

    return pl.pallas_call(
        kern,
        grid=(m // tn, n // tm),
        in_specs=[
            pl.BlockSpec((tm, kdim), lambda j, i: (i, 0)),
            pl.BlockSpec((kdim, tn), lambda j, i: (0, j)),
        ],
        out_specs=[
            pl.BlockSpec((tm, tn), lambda j, i: (i, j)),
            pl.BlockSpec((1, tn), lambda j, i: (0, j)),
        ],
        out_shape=[
            jax.ShapeDtypeStruct((n, m), jnp.bfloat16),
            jax.ShapeDtypeStruct((1, m), jnp.float32),
        ],
    )(a, b)


def _gcn_apply(A, xw, dinv, bias, tm=512, tk=512):
    n = A.shape[0]
    f = xw.shape[1]
    nk = n // tk
    d2 = dinv.reshape(1, n)
    b2 = bias.reshape(1, f)

    def kern(a_ref, s_ref, di_ref, dk_ref, xi_ref, b_ref, o_ref, acc_ref):
        k = pl.program_id(1)

        @pl.when(k == 0)
        def _():
            acc_ref[...] = jnp.zeros_like(acc_ref)

        a = a_ref[...].astype(jnp.float32)
        s = s_ref[...] * dk_ref[0, :][:, None]
        acc_ref[...] += jax.lax.dot_general(
            a, s, (((0,), (0,)), ((), ())), preferred_element_type=jnp.float32)

        @pl.when(k == nk - 1)
        def _():
            di = di_ref[0, :][:, None]
            o_ref[...] = jnp.maximum(
                di * acc_ref[...] + (di * di) * xi_ref[...] + b_ref[...], 0.0)

    return pl.pallas_call(
        kern,
        grid=(n // tm, nk),
        in_specs=[
            pl.BlockSpec((tk, tm), lambda i, k: (k, i)),
            pl.BlockSpec((tk, f), lambda i, k: (k, 0)),
            pl.BlockSpec((1, tm), lambda i, k: (0, i)),
            pl.BlockSpec((1, tk), lambda i, k: (0, k)),
            pl.BlockSpec((tm, f), lambda i, k: (i, 0)),
            pl.BlockSpec((1, f), lambda i, k: (0, 0)),
        ],
        out_specs=pl.BlockSpec((tm, f), lambda i, k: (i, 0)),
        out_shape=jax.ShapeDtypeStruct((n, f), jnp.float32),
        scratch_shapes=[pltpu.VMEM((tm, f), jnp.float32)],
    )(A, xw, d2, d2, xw, b2)


def _padf(m):
    pad = (-m.shape[-1]) % 128
    if pad == 0:
        return m
    if m.ndim == 1:
        return jnp.pad(m, (0, pad))
    return jnp.pad(m, ((0, 0), (0, pad)))


def _bn(h):
    m = h.mean(axis=0)
    v = h.var(axis=0)
    return (h - m) / jnp.sqrt(v + 1e-5)


def kernel(x, z, edge_index, batch, target, params):
    p = params
    n = x.shape[0]
    nb = target.shape[0]
    relu = jax.nn.relu

    zc = jnp.concatenate([z, x], axis=-1)
    h = relu(_bn(zc @ p['vae_W0'] + p['vae_b0']))
    h = _bn(h @ p['vae_W1'] + p['vae_b1'])
    aug = jnp.tanh(h)
    X = jnp.concatenate([aug, x], axis=1)

    src = edge_index[0]
    dst = edge_index[1]
    Wcnt = jnp.zeros((n, n), jnp.float32).at[src, dst].add(1.0)
    Wcnt = Wcnt.astype(jnp.bfloat16)
    deg_e = jnp.sum(Wcnt, axis=0, dtype=jnp.float32) + 1.0
    dinv_e = jax.lax.rsqrt(jnp.maximum(deg_e, 1.0))

    h1 = _gcn_apply(Wcnt, _padf(X @ p['gcn_W1']), dinv_e, _padf(p['gcn_b1']))[:, :156]
    h2 = _gcn_apply(Wcnt, _padf(h1 @ p['gcn_W2']), dinv_e, _padf(p['gcn_b2']))[:, :312]
    h3 = _gcn_apply(Wcnt, _padf(h2 @ p['gcn_W3']), dinv_e, _padf(p['gcn_b3']))[:, :624]

    a2, cs2 = _mm_thresh(Wcnt, Wcnt)
    dd2 = jax.lax.rsqrt(cs2[0] + 1.0)
    a3, cs3 = _mm_thresh(a2, Wcnt)
    dd3 = jax.lax.rsqrt(cs3[0] + 1.0)

    xw1 = _padf(X @ p['gcn_W1'])
    h4 = _gcn_apply(a2, xw1, dd2, _padf(p['gcn_b1']))[:, :156]
    h5 = _gcn_apply(a2, _padf(h4 @ p['gcn_W2']), dd2, _padf(p['gcn_b2']))[:, :312]
    h6 = _gcn_apply(a3, xw1, dd3, _padf(p['gcn_b1']))[:, :156]

    concat = jnp.concatenate([h3, h5, h6], axis=1)
    xg = jax.ops.segment_max(concat, batch, num_segments=nb)
    xg = relu(xg @ p['fcg1_W'] + p['fcg1_b'])
    xg = xg @ p['fcg2_W'] + p['fcg2_b']
    beta = jax.nn.softmax(jnp.tanh(xg), axis=1)
    xg = beta * xg

    emb = jnp.take(p['emb'], target, axis=0)
    e = jnp.transpose(emb, (0, 2, 1))

    def conv1d(xin, W, b):
        y = jax.lax.conv_general_dilated(
            xin, W, window_strides=(1,), padding='VALID',
            dimension_numbers=('NCH', 'OIH', 'NCH'))
        return relu(y + b[None, :, None])

    feats = []
    idx = 0
    for blk in range(3):
        hh = e
        for _ in range(blk + 1):
            hh = conv1d(hh, p['conv_W%d' % idx], p['conv_b%d' % idx])
            idx += 1
        feats.append(hh.max(axis=-1))
    xt = jnp.concatenate(feats, axis=-1) @ p['prot_W'] + p['prot_b']
    beta2 = jax.nn.softmax(jnp.tanh(xt), axis=1)
    xt = beta2 * xt

    xc = jnp.concatenate([xg, xt], axis=1)
    xc = relu(xc @ p['fc1_W'] + p['fc1_b'])
    xc = relu(xc @ p['fc2_W'] + p['fc2_b'])
    return xc @ p['out_W'] + p['out_b']


# speedup vs baseline: 3.8361x; 1.1276x over previous
import jax
import jax.numpy as jnp
from jax.experimental import pallas as pl
from jax.experimental.pallas import tpu as pltpu


def _mm_thresh(a, b, tm=1024, tn=1024):
    n, kdim = a.shape
    m = b.shape[1]

    def kern(a_ref, b_ref, o_ref, cs_ref):
        acc = jnp.dot(a_ref[...], b_ref[...], preferred_element_type=jnp.float32)
        t = (acc > 0.0).astype(jnp.float32)
        o_ref[...] = t.astype(jnp.bfloat16)
        part = jnp.sum(t, axis=0, keepdims=True)

        @pl.when(pl.program_id(1) == 0)
        def _():
            cs_ref[...] = part

        @pl.when(pl.program_id(1) != 0)
        def _():
            cs_ref[...] += part

    return pl.pallas_call(
        kern,
        grid=(m // tn, n // tm),
        in_specs=[
            pl.BlockSpec((tm, kdim), lambda j, i: (i, 0)),
            pl.BlockSpec((kdim, tn), lambda j, i: (0, j)),
        ],
        out_specs=[
            pl.BlockSpec((tm, tn), lambda j, i: (i, j)),
            pl.BlockSpec((1, tn), lambda j, i: (0, j)),
        ],
        out_shape=[
            jax.ShapeDtypeStruct((n, m), jnp.bfloat16),
            jax.ShapeDtypeStruct((1, m), jnp.float32),
        ],
    )(a, b)


def _gcn_apply(A, xw, dinv, bias, tm=1024, tk=512):
    n = A.shape[0]
    f = xw.shape[1]
    nk = n // tk
    d2 = dinv.reshape(1, n)
    b2 = bias.reshape(1, f)

    def kern(a_ref, s_ref, di_ref, dk_ref, xi_ref, b_ref, o_ref, acc_ref):
        k = pl.program_id(1)

        @pl.when(k == 0)
        def _():
            acc_ref[...] = jnp.zeros_like(acc_ref)

        a = a_ref[...].astype(jnp.float32)
        s = s_ref[...] * dk_ref[0, :][:, None]
        acc_ref[...] += jax.lax.dot_general(
            a, s, (((0,), (0,)), ((), ())), preferred_element_type=jnp.float32)

        @pl.when(k == nk - 1)
        def _():
            di = di_ref[0, :][:, None]
            o_ref[...] = jnp.maximum(
                di * acc_ref[...] + (di * di) * xi_ref[...] + b_ref[...], 0.0)

    return pl.pallas_call(
        kern,
        grid=(n // tm, nk),
        in_specs=[
            pl.BlockSpec((tk, tm), lambda i, k: (k, i)),
            pl.BlockSpec((tk, f), lambda i, k: (k, 0)),
            pl.BlockSpec((1, tm), lambda i, k: (0, i)),
            pl.BlockSpec((1, tk), lambda i, k: (0, k)),
            pl.BlockSpec((tm, f), lambda i, k: (i, 0)),
            pl.BlockSpec((1, f), lambda i, k: (0, 0)),
        ],
        out_specs=pl.BlockSpec((tm, f), lambda i, k: (i, 0)),
        out_shape=jax.ShapeDtypeStruct((n, f), jnp.float32),
        scratch_shapes=[pltpu.VMEM((tm, f), jnp.float32)],
    )(A, xw, d2, d2, xw, b2)


def _padf(m):
    pad = (-m.shape[-1]) % 128
    if pad == 0:
        return m
    if m.ndim == 1:
        return jnp.pad(m, (0, pad))
    return jnp.pad(m, ((0, 0), (0, pad)))


def _bn(h):
    m = h.mean(axis=0)
    v = h.var(axis=0)
    return (h - m) / jnp.sqrt(v + 1e-5)


def kernel(x, z, edge_index, batch, target, params):
    p = params
    n = x.shape[0]
    nb = target.shape[0]
    relu = jax.nn.relu

    zc = jnp.concatenate([z, x], axis=-1)
    h = relu(_bn(zc @ p['vae_W0'] + p['vae_b0']))
    h = _bn(h @ p['vae_W1'] + p['vae_b1'])
    aug = jnp.tanh(h)
    X = jnp.concatenate([aug, x], axis=1)

    src = edge_index[0]
    dst = edge_index[1]
    Wcnt = jnp.zeros((n, n), jnp.float32).at[src, dst].add(1.0)
    Wcnt = Wcnt.astype(jnp.bfloat16)
    deg_e = jnp.sum(Wcnt, axis=0, dtype=jnp.float32) + 1.0
    dinv_e = jax.lax.rsqrt(jnp.maximum(deg_e, 1.0))

    h1 = _gcn_apply(Wcnt, _padf(X @ p['gcn_W1']), dinv_e, _padf(p['gcn_b1']))[:, :156]
    h2 = _gcn_apply(Wcnt, _padf(h1 @ p['gcn_W2']), dinv_e, _padf(p['gcn_b2']))[:, :312]
    h3 = _gcn_apply(Wcnt, _padf(h2 @ p['gcn_W3']), dinv_e, _padf(p['gcn_b3']))[:, :624]

    a2, cs2 = _mm_thresh(Wcnt, Wcnt)
    dd2 = jax.lax.rsqrt(cs2[0] + 1.0)
    a3, cs3 = _mm_thresh(a2, Wcnt)
    dd3 = jax.lax.rsqrt(cs3[0] + 1.0)

    xw1 = _padf(X @ p['gcn_W1'])
    h4 = _gcn_apply(a2, xw1, dd2, _padf(p['gcn_b1']))[:, :156]
    h5 = _gcn_apply(a2, _padf(h4 @ p['gcn_W2']), dd2, _padf(p['gcn_b2']))[:, :312]
    h6 = _gcn_apply(a3, xw1, dd3, _padf(p['gcn_b1']))[:, :156]

    concat = jnp.concatenate([h3, h5, h6], axis=1)
    xg = jax.ops.segment_max(concat, batch, num_segments=nb)
    xg = relu(xg @ p['fcg1_W'] + p['fcg1_b'])
    xg = xg @ p['fcg2_W'] + p['fcg2_b']
    beta = jax.nn.softmax(jnp.tanh(xg), axis=1)
    xg = beta * xg

    emb = jnp.take(p['emb'], target, axis=0)
    e = jnp.transpose(emb, (0, 2, 1))

    def conv1d(xin, W, b):
        y = jax.lax.conv_general_dilated(
            xin, W, window_strides=(1,), padding='VALID',
            dimension_numbers=('NCH', 'OIH', 'NCH'))
        return relu(y + b[None, :, None])

    feats = []
    idx = 0
    for blk in range(3):
        hh = e
        for _ in range(blk + 1):
            hh = conv1d(hh, p['conv_W%d' % idx], p['conv_b%d' % idx])
            idx += 1
        feats.append(hh.max(axis=-1))
    xt = jnp.concatenate(feats, axis=-1) @ p['prot_W'] + p['prot_b']
    beta2 = jax.nn.softmax(jnp.tanh(xt), axis=1)
    xt = beta2 * xt

    xc = jnp.concatenate([xg, xt], axis=1)
    xc = relu(xc @ p['fc1_W'] + p['fc1_b'])
    xc = relu(xc @ p['fc2_W'] + p['fc2_b'])
    return xc @ p['out_W'] + p['out_b']


# speedup vs baseline: 3.9323x; 1.0251x over previous
import jax
import jax.numpy as jnp
from jax.experimental import pallas as pl
from jax.experimental.pallas import tpu as pltpu


def _mm_thresh(a, b, tm=1024, tn=1024):
    n, kdim = a.shape
    m = b.shape[1]

    def kern(a_ref, b_ref, o_ref, cs_ref):
        acc = jnp.dot(a_ref[...], b_ref[...], preferred_element_type=jnp.float32)
        t = (acc > 0.0).astype(jnp.float32)
        o_ref[...] = t.astype(jnp.bfloat16)
        part = jnp.sum(t, axis=0, keepdims=True)

        @pl.when(pl.program_id(1) == 0)
        def _():
            cs_ref[...] = part

        @pl.when(pl.program_id(1) != 0)
        def _():
            cs_ref[...] += part

    return pl.pallas_call(
        kern,
        grid=(m // tn, n // tm),
        in_specs=[
            pl.BlockSpec((tm, kdim), lambda j, i: (i, 0)),
            pl.BlockSpec((kdim, tn), lambda j, i: (0, j)),
        ],
        out_specs=[
            pl.BlockSpec((tm, tn), lambda j, i: (i, j)),
            pl.BlockSpec((1, tn), lambda j, i: (0, j)),
        ],
        out_shape=[
            jax.ShapeDtypeStruct((n, m), jnp.bfloat16),
            jax.ShapeDtypeStruct((1, m), jnp.float32),
        ],
    )(a, b)


def _gcn_apply(A, xw, dinv, bias, tm=1024, tk=512):
    n = A.shape[0]
    f = xw.shape[1]
    nk = n // tk
    d2 = dinv.reshape(1, n)
    b2 = bias.reshape(1, f)
    s_pre = (dinv[:, None] * xw).astype(jnp.bfloat16)

    def kern(a_ref, s_ref, di_ref, xi_ref, b_ref, o_ref, acc_ref):
        k = pl.program_id(1)

        @pl.when(k == 0)
        def _():
            acc_ref[...] = jnp.zeros_like(acc_ref)

        acc_ref[...] += jax.lax.dot_general(
            a_ref[...], s_ref[...], (((0,), (0,)), ((), ())),
            preferred_element_type=jnp.float32)

        @pl.when(k == nk - 1)
        def _():
            di = di_ref[0, :][:, None]
            o_ref[...] = jnp.maximum(
                di * acc_ref[...] + (di * di) * xi_ref[...] + b_ref[...], 0.0)

    return pl.pallas_call(
        kern,
        grid=(n // tm, nk),
        in_specs=[
            pl.BlockSpec((tk, tm), lambda i, k: (k, i)),
            pl.BlockSpec((tk, f), lambda i, k: (k, 0)),
            pl.BlockSpec((1, tm), lambda i, k: (0, i)),
            pl.BlockSpec((tm, f), lambda i, k: (i, 0)),
            pl.BlockSpec((1, f), lambda i, k: (0, 0)),
        ],
        out_specs=pl.BlockSpec((tm, f), lambda i, k: (i, 0)),
        out_shape=jax.ShapeDtypeStruct((n, f), jnp.float32),
        scratch_shapes=[pltpu.VMEM((tm, f), jnp.float32)],
    )(A, s_pre, d2, xw, b2)


def _padf(m):
    pad = (-m.shape[-1]) % 128
    if pad == 0:
        return m
    if m.ndim == 1:
        return jnp.pad(m, (0, pad))
    return jnp.pad(m, ((0, 0), (0, pad)))


def _bn(h):
    m = h.mean(axis=0)
    v = h.var(axis=0)
    return (h - m) / jnp.sqrt(v + 1e-5)


def kernel(x, z, edge_index, batch, target, params):
    p = params
    n = x.shape[0]
    nb = target.shape[0]
    relu = jax.nn.relu

    zc = jnp.concatenate([z, x], axis=-1)
    h = relu(_bn(zc @ p['vae_W0'] + p['vae_b0']))
    h = _bn(h @ p['vae_W1'] + p['vae_b1'])
    aug = jnp.tanh(h)
    X = jnp.concatenate([aug, x], axis=1)

    src = edge_index[0]
    dst = edge_index[1]
    Wcnt = jnp.zeros((n, n), jnp.float32).at[src, dst].add(1.0)
    Wcnt = Wcnt.astype(jnp.bfloat16)
    deg_e = jnp.sum(Wcnt, axis=0, dtype=jnp.float32) + 1.0
    dinv_e = jax.lax.rsqrt(jnp.maximum(deg_e, 1.0))

    h1 = _gcn_apply(Wcnt, _padf(X @ p['gcn_W1']), dinv_e, _padf(p['gcn_b1']))[:, :156]
    h2 = _gcn_apply(Wcnt, _padf(h1 @ p['gcn_W2']), dinv_e, _padf(p['gcn_b2']))[:, :312]
    h3 = _gcn_apply(Wcnt, _padf(h2 @ p['gcn_W3']), dinv_e, _padf(p['gcn_b3']))[:, :624]

    a2, cs2 = _mm_thresh(Wcnt, Wcnt)
    dd2 = jax.lax.rsqrt(cs2[0] + 1.0)
    a3, cs3 = _mm_thresh(a2, Wcnt)
    dd3 = jax.lax.rsqrt(cs3[0] + 1.0)

    xw1 = _padf(X @ p['gcn_W1'])
    h4 = _gcn_apply(a2, xw1, dd2, _padf(p['gcn_b1']))[:, :156]
    h5 = _gcn_apply(a2, _padf(h4 @ p['gcn_W2']), dd2, _padf(p['gcn_b2']))[:, :312]
    h6 = _gcn_apply(a3, xw1, dd3, _padf(p['gcn_b1']))[:, :156]

    concat = jnp.concatenate([h3, h5, h6], axis=1)
    xg = jax.ops.segment_max(concat, batch, num_segments=nb)
    xg = relu(xg @ p['fcg1_W'] + p['fcg1_b'])
    xg = xg @ p['fcg2_W'] + p['fcg2_b']
    beta = jax.nn.softmax(jnp.tanh(xg), axis=1)
    xg = beta * xg

    emb = jnp.take(p['emb'], target, axis=0)
    e = jnp.transpose(emb, (0, 2, 1))

    def conv1d(xin, W, b):
        y = jax.lax.conv_general_dilated(
            xin, W, window_strides=(1,), padding='VALID',
            dimension_numbers=('NCH', 'OIH', 'NCH'))
        return relu(y + b[None, :, None])

    feats = []
    idx = 0
    for blk in range(3):
        hh = e
        for _ in range(blk + 1):
            hh = conv1d(hh, p['conv_W%d' % idx], p['conv_b%d' % idx])
            idx += 1
        feats.append(hh.max(axis=-1))
    xt = jnp.concatenate(feats, axis=-1) @ p['prot_W'] + p['prot_b']
    beta2 = jax.nn.softmax(jnp.tanh(xt), axis=1)
    xt = beta2 * xt

    xc = jnp.concatenate([xg, xt], axis=1)
    xc = relu(xc @ p['fc1_W'] + p['fc1_b'])
    xc = relu(xc @ p['fc2_W'] + p['fc2_b'])
    return xc @ p['out_W'] + p['out_b']
